# Initial kernel scaffold; baseline (speedup 1.0000x reference)
import jax
import jax.numpy as jnp
from jax.experimental import pallas as pl


def kernel(x_site, x_vendor, edge_index, edge_label_index, W_site_in, b_site_in, W_vendor_in, b_vendor_in, W1sv_l, b1sv_l, W1sv_r, W1vs_l, b1vs_l, W1vs_r, W2sv_l, b2sv_l, W2sv_r, W2vs_l, b2vs_l, W2vs_r, Wd1, bd1, Wd2, bd2, Wd3, bd3):
    raise NotImplementedError("write your pallas kernel here")


# speedup vs baseline: 4.1870x; 4.1870x over previous
import functools

import jax
import jax.numpy as jnp
from jax import lax
from jax.experimental import pallas as pl
from jax.experimental.pallas import tpu as pltpu
from jax.experimental.pallas import tpu_sc as plsc

N = 10000
NP = 10240
E = 160000
B = 16384
D = 128
H = 256
CH = 128
NT = 16
ROWS_PER_TILE = NP // NT
CHUNKS_PER_TILE = -(-E // (CH * NT))
EP = CHUNKS_PER_TILE * CH * NT
F32 = jnp.float32


def _make_sc_aggregate():
    mesh = plsc.VectorSubcoreMesh(core_axis_name="c", subcore_axis_name="s")

    out_type = [
        jax.ShapeDtypeStruct((NP, D), F32),
        jax.ShapeDtypeStruct((NP, D), F32),
    ]
    scratch = [
        pltpu.VMEM_SHARED((NP, D), F32),
        pltpu.VMEM((CH,), jnp.int32),
        pltpu.VMEM((CH,), jnp.int32),
        pltpu.VMEM((CH, D), F32),
        pltpu.SemaphoreType.DMA,
    ]

    def body(t0_hbm, t1_hbm, g0_hbm, s0_hbm, g1_hbm, s1_hbm, z128_hbm,
             out0, out1, acc_sh, gidx_v, sidx_v, rows_v, sem):
        cid = lax.axis_index("c")
        sid = lax.axis_index("s")
        row0 = sid * ROWS_PER_TILE

        pltpu.sync_copy(z128_hbm, acc_sh.at[pl.ds(row0, ROWS_PER_TILE)])
        plsc.subcore_barrier()

        def direction(tbl, gsrc, ssrc):
            def chunk(j, carry):
                base = (j * NT + sid) * CH
                pltpu.sync_copy(gsrc.at[pl.ds(base, CH)], gidx_v)
                pltpu.sync_copy(ssrc.at[pl.ds(base, CH)], sidx_v)
                pltpu.async_copy(tbl.at[gidx_v], rows_v, sem).wait()
                pltpu.sync_copy(rows_v, acc_sh.at[sidx_v], add=True)
                return carry
            lax.fori_loop(0, CHUNKS_PER_TILE, chunk, 0)

        @pl.when(cid == 0)
        def _():
            direction(t0_hbm, g0_hbm, s0_hbm)

        @pl.when(cid == 1)
        def _():
            direction(t1_hbm, g1_hbm, s1_hbm)

        plsc.subcore_barrier()

        rows = pl.ds(row0, ROWS_PER_TILE)

        @pl.when(cid == 0)
        def _():
            pltpu.sync_copy(acc_sh.at[rows], out0.at[rows])

        @pl.when(cid == 1)
        def _():
            pltpu.sync_copy(acc_sh.at[rows], out1.at[rows])

    return pl.kernel(body, out_type=out_type, mesh=mesh, scratch_types=scratch,
                     name="sc_segsum")


_sc_aggregate = _make_sc_aggregate()


def _make_sc_counts():
    mesh = plsc.VectorSubcoreMesh(core_axis_name="c", subcore_axis_name="s")

    out_type = [
        jax.ShapeDtypeStruct((NP, D), F32),
        jax.ShapeDtypeStruct((NP, D), F32),
    ]
    scratch = [
        pltpu.VMEM_SHARED((NP, D), F32),
        pltpu.VMEM((CH,), jnp.int32),
        pltpu.VMEM((CH, D), F32),
    ]

    def body(s0_hbm, s1_hbm, z128_hbm, ones_hbm,
             out0, out1, acc_sh, sidx_v, ones_v):
        cid = lax.axis_index("c")
        sid = lax.axis_index("s")
        row0 = sid * ROWS_PER_TILE

        pltpu.sync_copy(z128_hbm, acc_sh.at[pl.ds(row0, ROWS_PER_TILE)])
        pltpu.sync_copy(ones_hbm, ones_v)
        plsc.subcore_barrier()

        def direction(ssrc):
            def chunk(j, carry):
                base = (j * NT + sid) * CH
                pltpu.sync_copy(ssrc.at[pl.ds(base, CH)], sidx_v)
                pltpu.sync_copy(ones_v, acc_sh.at[sidx_v], add=True)
                return carry
            lax.fori_loop(0, CHUNKS_PER_TILE, chunk, 0)

        @pl.when(cid == 0)
        def _():
            direction(s0_hbm)

        @pl.when(cid == 1)
        def _():
            direction(s1_hbm)

        plsc.subcore_barrier()

        rows = pl.ds(row0, ROWS_PER_TILE)

        @pl.when(cid == 0)
        def _():
            pltpu.sync_copy(acc_sh.at[rows], out0.at[rows])

        @pl.when(cid == 1)
        def _():
            pltpu.sync_copy(acc_sh.at[rows], out1.at[rows])

    return pl.kernel(body, out_type=out_type, mesh=mesh, scratch_types=scratch,
                     name="sc_counts")


_sc_counts = _make_sc_counts()


def _sc_decoder_gather():
    mesh = plsc.VectorSubcoreMesh(core_axis_name="c", subcore_axis_name="s")
    rows_per_tile = B // NT
    chunks = rows_per_tile // CH

    def body(t0_hbm, t1_hbm, i0_hbm, i1_hbm, out0, out1, idx_v, rows_v, sem):
        cid = lax.axis_index("c")
        sid = lax.axis_index("s")

        def run(tbl, isrc, out):
            def chunk(j, carry):
                base = (sid * chunks + j) * CH
                pltpu.sync_copy(isrc.at[pl.ds(base, CH)], idx_v)
                pltpu.async_copy(tbl.at[idx_v], rows_v, sem).wait()
                pltpu.sync_copy(rows_v, out.at[pl.ds(base, CH)])
                return carry
            lax.fori_loop(0, chunks, chunk, 0)

        @pl.when(cid == 0)
        def _():
            run(t0_hbm, i0_hbm, out0)

        @pl.when(cid == 1)
        def _():
            run(t1_hbm, i1_hbm, out1)

    return pl.kernel(
        body,
        out_type=[jax.ShapeDtypeStruct((B, D), F32),
                  jax.ShapeDtypeStruct((B, D), F32)],
        mesh=mesh,
        scratch_types=[pltpu.VMEM((CH,), jnp.int32),
                       pltpu.VMEM((CH, D), F32),
                       pltpu.SemaphoreType.DMA],
        name="sc_decoder_gather")


_decoder_gather = _sc_decoder_gather()


_RB = 1000


def _layer1_body(x_s, x_v, a_sv, c_v, a_vs, c_s,
                 Wsi, bsi, Wvi, bvi,
                 W1svl, b1svl, W1svr, W1vsl, b1vsl, W1vsr,
                 W2svl, W2vsl,
                 hv1_o, hs1_o, psv_o, pvs_o):
    dot = functools.partial(jnp.dot, preferred_element_type=F32)
    cntv = c_v[:, :1]
    cnts = c_s[:, :1]
    mean_sv = a_sv[:] / jnp.maximum(cntv, 1.0)
    mean_vs = a_vs[:] / jnp.maximum(cnts, 1.0)
    maskv = (cntv > 0.0).astype(F32)
    masks = (cnts > 0.0).astype(F32)
    hsm = dot(mean_sv, Wsi[:]) + maskv * bsi[:]
    hvm = dot(mean_vs, Wvi[:]) + masks * bvi[:]
    hv = dot(x_v[:], Wvi[:]) + bvi[:]
    hs = dot(x_s[:], Wsi[:]) + bsi[:]
    hv1 = jnp.maximum(dot(hsm, W1svl[:]) + b1svl[:] + dot(hv, W1svr[:]), 0.0)
    hs1 = jnp.maximum(dot(hvm, W1vsl[:]) + b1vsl[:] + dot(hs, W1vsr[:]), 0.0)
    hv1_o[:] = hv1
    hs1_o[:] = hs1
    psv_o[:] = dot(hs1, W2svl[:])
    pvs_o[:] = dot(hv1, W2vsl[:])


def _layer1_call(x_s, x_v, a_sv, c_v, a_vs, c_s, Wsi, bsi, Wvi, bvi,
                 W1svl, b1svl, W1svr, W1vsl, b1vsl, W1vsr, W2svl, W2vsl):
    g = N // _RB
    row = pl.BlockSpec((_RB, D), lambda i: (i, 0))
    full = lambda a: pl.BlockSpec(a.shape, lambda i: tuple(0 for _ in a.shape))
    return pl.pallas_call(
        _layer1_body,
        grid=(g,),
        in_specs=[row, row, row, row, row, row,
                  full(Wsi), full(bsi), full(Wvi), full(bvi),
                  full(W1svl), full(b1svl), full(W1svr),
                  full(W1vsl), full(b1vsl), full(W1vsr),
                  full(W2svl), full(W2vsl)],
        out_specs=[pl.BlockSpec((_RB, H), lambda i: (i, 0)),
                   pl.BlockSpec((_RB, H), lambda i: (i, 0)),
                   pl.BlockSpec((_RB, D), lambda i: (i, 0)),
                   pl.BlockSpec((_RB, D), lambda i: (i, 0))],
        out_shape=[jax.ShapeDtypeStruct((N, H), F32),
                   jax.ShapeDtypeStruct((N, H), F32),
                   jax.ShapeDtypeStruct((N, D), F32),
                   jax.ShapeDtypeStruct((N, D), F32)],
    )(x_s, x_v, a_sv, c_v, a_vs, c_s, Wsi, bsi, Wvi, bvi,
      W1svl, b1svl, W1svr, W1vsl, b1vsl, W1vsr, W2svl, W2vsl)


def _layer2_body(a2v, c_v, hv1, a2s, c_s, hs1,
                 b2svl, W2svr, b2vsl, W2vsr, hv2_o, hs2_o):
    dot = functools.partial(jnp.dot, preferred_element_type=F32)
    m2v = a2v[:] / jnp.maximum(c_v[:, :1], 1.0)
    m2s = a2s[:] / jnp.maximum(c_s[:, :1], 1.0)
    hv2_o[:] = jnp.maximum(m2v + b2svl[:] + dot(hv1[:], W2svr[:]), 0.0)
    hs2_o[:] = jnp.maximum(m2s + b2vsl[:] + dot(hs1[:], W2vsr[:]), 0.0)


def _layer2_call(a2v, c_v, hv1, a2s, c_s, hs1, b2svl, W2svr, b2vsl, W2vsr):
    g = N // _RB
    row = pl.BlockSpec((_RB, D), lambda i: (i, 0))
    hrow = pl.BlockSpec((_RB, H), lambda i: (i, 0))
    full = lambda a: pl.BlockSpec(a.shape, lambda i: tuple(0 for _ in a.shape))
    return pl.pallas_call(
        _layer2_body,
        grid=(g,),
        in_specs=[row, row, hrow, row, row, hrow,
                  full(b2svl), full(W2svr), full(b2vsl), full(W2vsr)],
        out_specs=[pl.BlockSpec((_RB, D), lambda i: (i, 0)),
                   pl.BlockSpec((_RB, D), lambda i: (i, 0))],
        out_shape=[jax.ShapeDtypeStruct((N, D), F32),
                   jax.ShapeDtypeStruct((N, D), F32)],
    )(a2v, c_v, hv1, a2s, c_s, hs1, b2svl, W2svr, b2vsl, W2vsr)


_DB = 2048


def _decoder_body(zs, zv, Wd1a, Wd1b, bd1, Wd2, bd2, Wd3, bd3, out_o):
    dot = functools.partial(jnp.dot, preferred_element_type=F32)
    z = jnp.maximum(dot(zs[:], Wd1a[:]) + dot(zv[:], Wd1b[:]) + bd1[:], 0.0)
    z = jnp.maximum(dot(z, Wd2[:]) + bd2[:], 0.0)
    out_o[:] = jax.nn.sigmoid(dot(z, Wd3[:]) + bd3[:])


def _decoder_call(zs, zv, Wd1a, Wd1b, bd1, Wd2, bd2, Wd3, bd3):
    g = B // _DB
    row = pl.BlockSpec((_DB, D), lambda i: (i, 0))
    full = lambda a: pl.BlockSpec(a.shape, lambda i: tuple(0 for _ in a.shape))
    return pl.pallas_call(
        _decoder_body,
        grid=(g,),
        in_specs=[row, row, full(Wd1a), full(Wd1b), full(bd1),
                  full(Wd2), full(bd2), full(Wd3), full(bd3)],
        out_specs=pl.BlockSpec((_DB, 1), lambda i: (i, 0)),
        out_shape=jax.ShapeDtypeStruct((B, 1), F32),
    )(zs, zv, Wd1a, Wd1b, bd1, Wd2, bd2, Wd3, bd3)


def _dbg_kernel_agg_only(x_site, x_vendor, edge_index, edge_label_index,
           W_site_in, b_site_in, W_vendor_in, b_vendor_in,
           W1sv_l, b1sv_l, W1sv_r, W1vs_l, b1vs_l, W1vs_r,
           W2sv_l, b2sv_l, W2sv_r, W2vs_l, b2vs_l, W2vs_r,
           Wd1, bd1, Wd2, bd2, Wd3, bd3):
    s0, v0 = edge_index[0], edge_index[1]
    cnt_v1 = jax.ops.segment_sum(jnp.ones((E,), F32), v0, num_segments=N)
    cnt_s1 = jax.ops.segment_sum(jnp.ones((E,), F32), s0, num_segments=N)
    cnt_v = jnp.broadcast_to(cnt_v1[:, None], (N, D))
    cnt_s = jnp.broadcast_to(cnt_s1[:, None], (N, D))
    agg_sv = jax.ops.segment_sum(x_site[s0], v0, num_segments=N)
    agg_vs = jax.ops.segment_sum(x_vendor[v0], s0, num_segments=N)

    r = lambda b: b.reshape(1, -1)
    hv1, hs1, p_sv, p_vs = _layer1_call(
        x_site, x_vendor, agg_sv, cnt_v, agg_vs, cnt_s,
        W_site_in, r(b_site_in), W_vendor_in, r(b_vendor_in),
        W1sv_l, r(b1sv_l), W1sv_r, W1vs_l, r(b1vs_l), W1vs_r,
        W2sv_l, W2vs_l)

    agg2_v = jax.ops.segment_sum(p_sv[s0], v0, num_segments=N)
    agg2_s = jax.ops.segment_sum(p_vs[v0], s0, num_segments=N)

    hv2, hs2 = _layer2_call(agg2_v, cnt_v, hv1, agg2_s, cnt_s, hs1,
                            r(b2sv_l), W2sv_r, r(b2vs_l), W2vs_r)

    zs = hs2[edge_label_index[0]]
    zv = hv2[edge_label_index[1]]
    return _decoder_call(zs, zv, Wd1[:D], Wd1[D:], r(bd1),
                         Wd2, r(bd2), Wd3, r(bd3))


def kernel(x_site, x_vendor, edge_index, edge_label_index,
           W_site_in, b_site_in, W_vendor_in, b_vendor_in,
           W1sv_l, b1sv_l, W1sv_r, W1vs_l, b1vs_l, W1vs_r,
           W2sv_l, b2sv_l, W2sv_r, W2vs_l, b2vs_l, W2vs_r,
           Wd1, bd1, Wd2, bd2, Wd3, bd3):
    pad_g = jnp.zeros((EP - E,), jnp.int32)
    pad_s = jnp.full((EP - E,), NP - 1, jnp.int32)
    s = jnp.concatenate([edge_index[0], pad_g])
    v = jnp.concatenate([edge_index[1], pad_g])
    s_sc = jnp.concatenate([edge_index[0], pad_s])
    v_sc = jnp.concatenate([edge_index[1], pad_s])
    eli0 = edge_label_index[0]
    eli1 = edge_label_index[1]

    z128 = jnp.zeros((ROWS_PER_TILE, D), F32)
    ones128 = jnp.ones((CH, D), F32)

    cnt_v, cnt_s = _sc_counts(v_sc, s_sc, z128, ones128)

    cslice = cnt_v[:ROWS_PER_TILE, :D]
    z128a = jnp.where(cslice < 0.0, cslice, z128)

    agg_sv, agg_vs = _sc_aggregate(x_site, x_vendor, s, v_sc, v, s_sc, z128a)

    r = lambda b: b.reshape(1, -1)
    hv1, hs1, p_sv, p_vs = _layer1_call(
        x_site, x_vendor, agg_sv, cnt_v, agg_vs, cnt_s,
        W_site_in, r(b_site_in), W_vendor_in, r(b_vendor_in),
        W1sv_l, r(b1sv_l), W1sv_r, W1vs_l, r(b1vs_l), W1vs_r,
        W2sv_l, W2vs_l)

    agg2_v, agg2_s = _sc_aggregate(p_sv, p_vs, s, v_sc, v, s_sc, z128)

    hv2, hs2 = _layer2_call(agg2_v, cnt_v, hv1, agg2_s, cnt_s, hs1,
                            r(b2sv_l), W2sv_r, r(b2vs_l), W2vs_r)

    zs, zv = _decoder_gather(hs2, hv2, eli0, eli1)

    return _decoder_call(zs, zv, Wd1[:D], Wd1[D:], r(bd1),
                         Wd2, r(bd2), Wd3, r(bd3))
